# baseline (speedup 1.0000x reference)
import functools

import jax
import jax.numpy as jnp
from jax import lax
from jax.experimental import pallas as pl
from jax.experimental.pallas import tpu as pltpu
from jax.experimental.pallas import tpu_sc as plsc

NC = 2
NS = 16
NW = NC * NS
IDXW = 128
K = 6
F = 16

_mesh = functools.partial(
    plsc.VectorSubcoreMesh, core_axis_name="c", subcore_axis_name="s",
    num_cores=NC, num_subcores=NS)


def _seg_sum_kernel(n_pad, blocks, zero_chunk, with_gather):
  stripe = n_pad // NS
  reps = stripe // zero_chunk
  assert blocks % 4 == 0

  def body(*refs):
    if with_gather:
      (y_hbm, src2d, dst2d, zeros_hbm, drain_hbm, out) = refs[:6]
      src_v = refs[6:8]
      dst_v = refs[8:10]
      rows_v = refs[10:12]
      zero_v, acc = refs[12:14]
      gsem = refs[14:16]
      ssem = refs[16:18]
    else:
      (ones_hbm, dst2d, zeros_hbm, out,
       dst0, dst1, ones_v, zero_v, acc, ssem0, ssem1) = refs
      drain_hbm = ones_hbm
      dst_v = (dst0, dst1)
      ssem = (ssem0, ssem1)
    cid = lax.axis_index("c")
    tid = lax.axis_index("s")
    wid = cid * NS + tid

    pltpu.sync_copy(zeros_hbm, zero_v)

    @pl.loop(0, reps)
    def _(i):
      pltpu.sync_copy(zero_v, acc.at[pl.ds(tid * stripe + i * zero_chunk,
                                           zero_chunk)])

    if not with_gather:
      pltpu.sync_copy(ones_hbm, ones_v)
    plsc.subcore_barrier()

    row_base = wid * blocks * K

    def load_idx(b, p):
      if with_gather:
        pltpu.sync_copy(src2d.at[pl.ds(row_base + b * K, K)], src_v[p])
      pltpu.sync_copy(dst2d.at[pl.ds(row_base + b * K, K)], dst_v[p])

    def start_gather(p):
      for j in range(K):
        pltpu.async_copy(y_hbm.at[src_v[p].at[j]], rows_v[p].at[j], gsem[p])

    def drain_gather(p):
      pltpu.make_async_copy(drain_hbm, rows_v[p], gsem[p]).wait()

    def start_scatter(p):
      for j in range(K):
        rows = rows_v[p].at[j] if with_gather else ones_v.at[j]
        pltpu.async_copy(rows, acc.at[dst_v[p].at[j]], ssem[p], add=True)

    def drain_scatter(p):
      sink = rows_v[p] if with_gather else ones_v
      pltpu.make_async_copy(drain_hbm, sink, ssem[p]).wait()

    if with_gather:
      load_idx(0, 0)
      start_gather(0)
      load_idx(1, 1)
      start_gather(1)

      @pl.loop(0, blocks // 2 - 1)
      def _(g):
        b = g * 2
        for p in range(2):
          drain_gather(p)
          start_scatter(p)
          drain_scatter(p)
          load_idx(b + 2 + p, p)
          start_gather(p)

      for p in range(2):
        drain_gather(p)
        start_scatter(p)
        drain_scatter(p)
    else:
      load_idx(0, 0)
      start_scatter(0)
      load_idx(1, 1)
      start_scatter(1)

      @pl.loop(0, blocks // 2 - 1)
      def _(g):
        b = g * 2
        for p in range(2):
          drain_scatter(p)
          load_idx(b + 2 + p, p)
          start_scatter(p)

      for p in range(2):
        drain_scatter(p)

    plsc.subcore_barrier()
    pltpu.sync_copy(acc.at[pl.ds(tid * stripe, stripe)],
                    out.at[cid, pl.ds(tid * stripe, stripe)])

  scratch = []
  if with_gather:
    scratch += [pltpu.VMEM((K, IDXW), jnp.int32)] * 2
    scratch += [pltpu.VMEM((K, IDXW), jnp.int32)] * 2
    scratch += [pltpu.VMEM((K, IDXW, F), jnp.float32)] * 2
  else:
    scratch += [pltpu.VMEM((K, IDXW), jnp.int32)] * 2
    scratch += [pltpu.VMEM((K, IDXW, F), jnp.float32)]
  scratch += [
      pltpu.VMEM((zero_chunk, F), jnp.float32),
      pltpu.VMEM_SHARED((n_pad, F), jnp.float32),
  ]
  if with_gather:
    scratch += [pltpu.SemaphoreType.DMA] * 2
    scratch += [pltpu.SemaphoreType.DMA] * 2
  else:
    scratch += [pltpu.SemaphoreType.DMA] * 2
  return pl.kernel(
      body,
      out_type=jax.ShapeDtypeStruct((NC, n_pad, F), jnp.float32),
      mesh=_mesh(),
      scratch_types=scratch,
      compiler_params=pltpu.CompilerParams(use_tc_tiling_on_sc=False),
  )


def _tc1_body(s1p, dgp, xv, bd_l1, b1, bd_r1, bd_l2, h_ref, y2_ref):
  s = s1p[0] + s1p[1]
  deg = jnp.maximum(dgp[0] + dgp[1], 1.0)
  mean = s / deg
  dn = (((1,), (0,)), ((), ()))
  h = (lax.dot_general(mean, bd_l1[...], dn,
                       preferred_element_type=jnp.float32)
       + b1[...]
       + lax.dot_general(xv[...], bd_r1[...], dn,
                         preferred_element_type=jnp.float32))
  h = jnp.maximum(h, 0.0)
  h_ref[...] = h
  y2_ref[...] = lax.dot_general(h, bd_l2[...], dn,
                                preferred_element_type=jnp.float32)


def _tc2_body(s2p, dgp, h, bd_r2, b2, out_ref):
  s = s2p[0] + s2p[1]
  deg = jnp.maximum(dgp[0] + dgp[1], 1.0)
  dn = (((1,), (0,)), ((), ()))
  out_ref[...] = (s / deg + b2[...]
                  + lax.dot_general(h[...], bd_r2[...], dn,
                                    preferred_element_type=jnp.float32))


@jax.jit
def kernel(x, edge_index, W_l1, b_l1, W_r1, W_l2, b_l2, W_r2):
  n = x.shape[0]
  e = edge_index.shape[1]

  per_block = NW * K * IDXW
  blocks = -(-e // per_block)
  blocks += (-blocks) % 4
  e_pad = blocks * per_block
  n_pad = -(-(n + 1) // 128) * 128
  zero_chunk = n_pad // NS // 72

  src = edge_index[0]
  dst = edge_index[1]
  pad = e_pad - e
  if pad:
    src = jnp.concatenate([src, jnp.zeros((pad,), jnp.int32)])
    dst = jnp.concatenate([dst, jnp.full((pad,), n, jnp.int32)])
  src2d = src.reshape(e_pad // IDXW, IDXW)
  dst2d = dst.reshape(e_pad // IDXW, IDXW)

  zeros_hbm = jnp.zeros((zero_chunk, F), jnp.float32)
  ones_hbm = jnp.ones((K, IDXW, F), jnp.float32)

  deg16 = _seg_sum_kernel(n_pad, blocks, zero_chunk, False)(
      ones_hbm, dst2d, zeros_hbm)
  s1 = _seg_sum_kernel(n_pad, blocks, zero_chunk, True)(
      x, src2d, dst2d, zeros_hbm, ones_hbm)

  n8 = n // 8
  np8 = n_pad // 8
  s1v = s1.reshape(NC, np8, 128)
  degv = deg16.reshape(NC, np8, 128)
  xv = x.reshape(n8, 128)
  eye8 = jnp.eye(8, dtype=jnp.float32)
  bd_l1 = jnp.kron(eye8, W_l1.T)
  bd_r1 = jnp.kron(eye8, W_r1.T)
  bd_l2 = jnp.kron(eye8, W_l2.T)
  bd_r2 = jnp.kron(eye8, W_r2.T)
  b1t = jnp.tile(b_l1, 8).reshape(1, 256)
  b2t = jnp.tile(b_l2, 8).reshape(1, 128)

  bn = 1000
  grid = (-(-n8 // bn),)
  full2 = pl.BlockSpec((NC, bn, 128), lambda i: (0, i, 0))
  rows = lambda w: pl.BlockSpec((bn, w), lambda i: (i, 0))
  const = lambda shape: pl.BlockSpec(shape, lambda i: tuple(0 for _ in shape))

  h, y2 = pl.pallas_call(
      _tc1_body,
      grid=grid,
      in_specs=[full2, full2, rows(128),
                const((128, 256)), const((1, 256)), const((128, 256)),
                const((256, 128))],
      out_specs=[rows(256), rows(128)],
      out_shape=[jax.ShapeDtypeStruct((n8, 256), jnp.float32),
                 jax.ShapeDtypeStruct((n8, 128), jnp.float32)],
  )(s1v, degv, xv, bd_l1, b1t, bd_r1, bd_l2)

  s2 = _seg_sum_kernel(n_pad, blocks, zero_chunk, True)(
      y2.reshape(n, F), src2d, dst2d, zeros_hbm, ones_hbm)

  out = pl.pallas_call(
      _tc2_body,
      grid=grid,
      in_specs=[full2, full2, rows(256),
                const((256, 128)), const((1, 128))],
      out_specs=rows(128),
      out_shape=jax.ShapeDtypeStruct((n8, 128), jnp.float32),
  )(s2.reshape(NC, np8, 128), degv, h, bd_r2, b2t)
  return out.reshape(n, F)


# speedup vs baseline: 1.2316x; 1.2316x over previous
import functools

import jax
import jax.numpy as jnp
from jax import lax
from jax.experimental import pallas as pl
from jax.experimental.pallas import tpu as pltpu
from jax.experimental.pallas import tpu_sc as plsc

NC = 2
NS = 16
NW = NC * NS
IDXW = 128
K = 4
F = 16

_mesh = functools.partial(
    plsc.VectorSubcoreMesh, core_axis_name="c", subcore_axis_name="s",
    num_cores=NC, num_subcores=NS)


def _seg_sum_kernel(n_pad, blocks, zero_chunk, with_gather):
  stripe = n_pad // NS
  reps = stripe // zero_chunk
  assert blocks % 4 == 0

  def body(*refs):
    if with_gather:
      (y_hbm, src2d, dst2d, zeros_hbm, drain_hbm, out) = refs[:6]
      src_v = refs[6:8]
      dst_v = refs[8:10]
      rows_v = refs[10:12]
      zero_v, acc = refs[12:14]
      gsem = refs[14:16]
      ssem = refs[16:18]
    else:
      (ones_hbm, dst2d, zeros_hbm, out,
       dst0, dst1, ones_v, zero_v, acc, ssem0, ssem1) = refs
      drain_hbm = ones_hbm
      dst_v = (dst0, dst1)
      ssem = (ssem0, ssem1)
    cid = lax.axis_index("c")
    tid = lax.axis_index("s")
    wid = cid * NS + tid

    pltpu.sync_copy(zeros_hbm, zero_v)

    @pl.loop(0, reps)
    def _(i):
      pltpu.sync_copy(zero_v, acc.at[pl.ds(tid * stripe + i * zero_chunk,
                                           zero_chunk)])

    if not with_gather:
      pltpu.sync_copy(ones_hbm, ones_v)
    plsc.subcore_barrier()

    row_base = wid * blocks * K

    def load_idx(b, p):
      if with_gather:
        pltpu.sync_copy(src2d.at[pl.ds(row_base + b * K, K)], src_v[p])
      pltpu.sync_copy(dst2d.at[pl.ds(row_base + b * K, K)], dst_v[p])

    def start_gather(p):
      for j in range(K):
        pltpu.async_copy(y_hbm.at[src_v[p].at[j]], rows_v[p].at[j], gsem[p])

    def drain_gather(p):
      pltpu.make_async_copy(drain_hbm, rows_v[p], gsem[p]).wait()

    def start_scatter(p):
      for j in range(K):
        rows = rows_v[p].at[j] if with_gather else ones_v.at[j]
        pltpu.async_copy(rows, acc.at[dst_v[p].at[j]], ssem[p], add=True)

    def drain_scatter(p):
      sink = rows_v[p] if with_gather else ones_v
      pltpu.make_async_copy(drain_hbm, sink, ssem[p]).wait()

    if with_gather:
      load_idx(0, 0)
      start_gather(0)
      load_idx(1, 1)
      start_gather(1)

      @pl.loop(0, blocks // 2 - 1)
      def _(g):
        b = g * 2
        for p in range(2):
          drain_gather(p)
          start_scatter(p)
          drain_scatter(p)
          load_idx(b + 2 + p, p)
          start_gather(p)

      for p in range(2):
        drain_gather(p)
        start_scatter(p)
        drain_scatter(p)
    else:
      load_idx(0, 0)
      start_scatter(0)
      load_idx(1, 1)
      start_scatter(1)

      @pl.loop(0, blocks // 2 - 1)
      def _(g):
        b = g * 2
        for p in range(2):
          drain_scatter(p)
          load_idx(b + 2 + p, p)
          start_scatter(p)

      for p in range(2):
        drain_scatter(p)

    plsc.subcore_barrier()
    pltpu.sync_copy(acc.at[pl.ds(tid * stripe, stripe)],
                    out.at[cid, pl.ds(tid * stripe, stripe)])

  scratch = []
  if with_gather:
    scratch += [pltpu.VMEM((K, IDXW), jnp.int32)] * 2
    scratch += [pltpu.VMEM((K, IDXW), jnp.int32)] * 2
    scratch += [pltpu.VMEM((K, IDXW, F), jnp.float32)] * 2
  else:
    scratch += [pltpu.VMEM((K, IDXW), jnp.int32)] * 2
    scratch += [pltpu.VMEM((K, IDXW, F), jnp.float32)]
  scratch += [
      pltpu.VMEM((zero_chunk, F), jnp.float32),
      pltpu.VMEM_SHARED((n_pad, F), jnp.float32),
  ]
  if with_gather:
    scratch += [pltpu.SemaphoreType.DMA] * 2
    scratch += [pltpu.SemaphoreType.DMA] * 2
  else:
    scratch += [pltpu.SemaphoreType.DMA] * 2
  return pl.kernel(
      body,
      out_type=jax.ShapeDtypeStruct((NC, n_pad, F), jnp.float32),
      mesh=_mesh(),
      scratch_types=scratch,
      compiler_params=pltpu.CompilerParams(use_tc_tiling_on_sc=False),
  )


def _tc1_body(s1p, dgp, xv, bd_l1, b1, bd_r1, bd_l2, h_ref, y2_ref):
  s = s1p[0] + s1p[1]
  deg = jnp.maximum(dgp[0] + dgp[1], 1.0)
  mean = s / deg
  dn = (((1,), (0,)), ((), ()))
  h = (lax.dot_general(mean, bd_l1[...], dn,
                       preferred_element_type=jnp.float32)
       + b1[...]
       + lax.dot_general(xv[...], bd_r1[...], dn,
                         preferred_element_type=jnp.float32))
  h = jnp.maximum(h, 0.0)
  h_ref[...] = h
  y2_ref[...] = lax.dot_general(h, bd_l2[...], dn,
                                preferred_element_type=jnp.float32)


def _tc2_body(s2p, dgp, h, bd_r2, b2, out_ref):
  s = s2p[0] + s2p[1]
  deg = jnp.maximum(dgp[0] + dgp[1], 1.0)
  dn = (((1,), (0,)), ((), ()))
  out_ref[...] = (s / deg + b2[...]
                  + lax.dot_general(h[...], bd_r2[...], dn,
                                    preferred_element_type=jnp.float32))


@jax.jit
def kernel(x, edge_index, W_l1, b_l1, W_r1, W_l2, b_l2, W_r2):
  n = x.shape[0]
  e = edge_index.shape[1]

  per_block = NW * K * IDXW
  blocks = -(-e // per_block)
  blocks += (-blocks) % 4
  e_pad = blocks * per_block
  n_pad = -(-(n + 1) // 128) * 128
  zero_chunk = n_pad // NS // 8

  src = edge_index[0]
  dst = edge_index[1]
  pad = e_pad - e
  if pad:
    src = jnp.concatenate([src, jnp.zeros((pad,), jnp.int32)])
    dst = jnp.concatenate([dst, jnp.full((pad,), n, jnp.int32)])
  src2d = src.reshape(e_pad // IDXW, IDXW)
  dst2d = dst.reshape(e_pad // IDXW, IDXW)

  zeros_hbm = jnp.zeros((zero_chunk, F), jnp.float32)
  ones_hbm = jnp.ones((K, IDXW, F), jnp.float32)

  deg16 = _seg_sum_kernel(n_pad, blocks, zero_chunk, False)(
      ones_hbm, dst2d, zeros_hbm)
  s1 = _seg_sum_kernel(n_pad, blocks, zero_chunk, True)(
      x, src2d, dst2d, zeros_hbm, ones_hbm)

  n8 = n // 8
  np8 = n_pad // 8
  s1v = s1.reshape(NC, np8, 128)
  degv = deg16.reshape(NC, np8, 128)
  xv = x.reshape(n8, 128)
  eye8 = jnp.eye(8, dtype=jnp.float32)
  bd_l1 = jnp.kron(eye8, W_l1.T)
  bd_r1 = jnp.kron(eye8, W_r1.T)
  bd_l2 = jnp.kron(eye8, W_l2.T)
  bd_r2 = jnp.kron(eye8, W_r2.T)
  b1t = jnp.tile(b_l1, 8).reshape(1, 256)
  b2t = jnp.tile(b_l2, 8).reshape(1, 128)

  bn = 1000
  grid = (-(-n8 // bn),)
  full2 = pl.BlockSpec((NC, bn, 128), lambda i: (0, i, 0))
  rows = lambda w: pl.BlockSpec((bn, w), lambda i: (i, 0))
  const = lambda shape: pl.BlockSpec(shape, lambda i: tuple(0 for _ in shape))

  h, y2 = pl.pallas_call(
      _tc1_body,
      grid=grid,
      in_specs=[full2, full2, rows(128),
                const((128, 256)), const((1, 256)), const((128, 256)),
                const((256, 128))],
      out_specs=[rows(256), rows(128)],
      out_shape=[jax.ShapeDtypeStruct((n8, 256), jnp.float32),
                 jax.ShapeDtypeStruct((n8, 128), jnp.float32)],
  )(s1v, degv, xv, bd_l1, b1t, bd_r1, bd_l2)

  s2 = _seg_sum_kernel(n_pad, blocks, zero_chunk, True)(
      y2.reshape(n, F), src2d, dst2d, zeros_hbm, ones_hbm)

  out = pl.pallas_call(
      _tc2_body,
      grid=grid,
      in_specs=[full2, full2, rows(256),
                const((256, 128)), const((1, 128))],
      out_specs=rows(128),
      out_shape=jax.ShapeDtypeStruct((n8, 128), jnp.float32),
  )(s2.reshape(NC, np8, 128), degv, h, bd_r2, b2t)
  return out.reshape(n, F)
